# baseline (speedup 1.0000x reference)
import functools

import jax
import jax.numpy as jnp
from jax import lax
from jax.experimental import pallas as pl
from jax.experimental.pallas import tpu as pltpu
from jax.experimental.pallas import tpu_sc as plsc

N = 10000
H = 128
O = 64
E = 320000

NC = 2
NS = 16
L = 16

EPC = E // NC
EPT = EPC // NS
CH = 40
NCHUNK = EPT // CH
NBUF = 5

NPAD = 10240
STRIPE = NPAD // NS

_mesh = plsc.VectorSubcoreMesh(core_axis_name="c", subcore_axis_name="s",
                               num_cores=NC, num_subcores=NS)


@functools.partial(
    pl.kernel,
    out_type=jax.ShapeDtypeStruct((NC, NPAD), jnp.float32),
    mesh=_mesh,
    scratch_types=[
        pltpu.VMEM_SHARED((NPAD,), jnp.float32),
        pltpu.VMEM((EPT,), jnp.int32),
        pltpu.VMEM((EPT,), jnp.float32),
        pltpu.VMEM((STRIPE,), jnp.float32),
        pltpu.SemaphoreType.DMA,
    ],
)
def _deg_kernel(dst_hbm, ew_hbm, out_hbm, acc, idx_b, val_b, zbuf, sem):
    c = lax.axis_index("c")
    s = lax.axis_index("s")
    base = c * EPC + s * EPT

    pltpu.async_copy(dst_hbm.at[pl.ds(base, EPT)], idx_b, sem)
    pltpu.async_copy(ew_hbm.at[pl.ds(base, EPT)], val_b, sem)

    @plsc.parallel_loop(0, STRIPE // L, unroll=8)
    def zero_body(j):
        zbuf[pl.ds(j * L, L)] = jnp.zeros((L,), jnp.float32)
    pltpu.sync_copy(zbuf, acc.at[pl.ds(s * STRIPE, STRIPE)])

    pltpu.make_async_copy(dst_hbm.at[pl.ds(base, EPT)], idx_b, sem).wait()
    pltpu.make_async_copy(ew_hbm.at[pl.ds(base, EPT)], val_b, sem).wait()
    plsc.subcore_barrier()

    pltpu.sync_copy(val_b, acc.at[idx_b], add=True)

    plsc.subcore_barrier()
    pltpu.sync_copy(acc.at[pl.ds(s * STRIPE, STRIPE)],
                    out_hbm.at[c, pl.ds(s * STRIPE, STRIPE)])


@functools.partial(
    pl.kernel,
    out_type=jax.ShapeDtypeStruct((NC, NPAD, H), jnp.float32),
    mesh=_mesh,
    scratch_types=[
        pltpu.VMEM_SHARED((NPAD, H), jnp.float32),
        pltpu.VMEM((EPT,), jnp.int32),
        pltpu.VMEM((EPT,), jnp.float32),
        pltpu.VMEM((CH,), jnp.int32),
        pltpu.VMEM((CH,), jnp.int32),
        pltpu.VMEM((CH,), jnp.int32),
        pltpu.VMEM((CH,), jnp.int32),
        pltpu.VMEM((CH,), jnp.int32),
        pltpu.VMEM((CH, H), jnp.float32),
        pltpu.VMEM((CH, H), jnp.float32),
        pltpu.VMEM((CH, H), jnp.float32),
        pltpu.VMEM((CH, H), jnp.float32),
        pltpu.VMEM((CH, H), jnp.float32),
        pltpu.SemaphoreType.DMA,
        pltpu.SemaphoreType.DMA,
        pltpu.SemaphoreType.DMA,
        pltpu.SemaphoreType.DMA,
        pltpu.SemaphoreType.DMA,
        pltpu.SemaphoreType.DMA,
        pltpu.SemaphoreType.DMA,
        pltpu.SemaphoreType.DMA,
        pltpu.SemaphoreType.DMA,
        pltpu.SemaphoreType.DMA,
    ],
    compiler_params=pltpu.CompilerParams(needs_layout_passes=False),
)
def _spmm_kernel(src_hbm, dst_hbm, ew_hbm, xs_hbm, out_hbm,
                 acc, sidx, ewb, didx0, didx1, didx2, didx3, didx4,
                 rows0, rows1, rows2, rows3, rows4,
                 g0, g1, g2, g3, g4, s0, s1, s2, s3, s4):
    c = lax.axis_index("c")
    s = lax.axis_index("s")
    gsem = (g0, g1, g2, g3, g4)
    ssem = (s0, s1, s2, s3, s4)
    didxs = (didx0, didx1, didx2, didx3, didx4)
    rows = (rows0, rows1, rows2, rows3, rows4)

    base = c * EPC + s * EPT
    pltpu.async_copy(src_hbm.at[pl.ds(base, EPT)], sidx, g0)
    pltpu.async_copy(ew_hbm.at[pl.ds(base, EPT)], ewb, g1)

    @plsc.parallel_loop(0, CH * (H // L), unroll=8)
    def zrow(j):
        k = j // (H // L)
        col = (j % (H // L)) * L
        rows0[k, pl.ds(col, L)] = jnp.zeros((L,), jnp.float32)

    def zcopy(r, _):
        pltpu.async_copy(rows0.at[pl.ds(0, CH)],
                         acc.at[pl.ds(s * STRIPE + r * CH, CH)], s0)
        return 0
    lax.fori_loop(0, STRIPE // CH, zcopy, 0)

    def zdrain(r, _):
        pltpu.make_async_copy(rows0.at[pl.ds(0, CH)],
                              acc.at[pl.ds(s * STRIPE, CH)], s0).wait()
        return 0
    lax.fori_loop(0, STRIPE // CH, zdrain, 0)

    pltpu.make_async_copy(src_hbm.at[pl.ds(base, EPT)], sidx, g0).wait()
    pltpu.make_async_copy(ew_hbm.at[pl.ds(base, EPT)], ewb, g1).wait()
    plsc.subcore_barrier()

    def gather(i, b):
        pltpu.async_copy(dst_hbm.at[pl.ds(base + i * CH, CH)],
                         didxs[b], gsem[b])
        pltpu.async_copy(xs_hbm.at[sidx.at[pl.ds(i * CH, CH)]],
                         rows[b], gsem[b])

    def gwait(i, b):
        pltpu.make_async_copy(dst_hbm.at[pl.ds(base + i * CH, CH)],
                              didxs[b], gsem[b]).wait()
        pltpu.make_async_copy(xs_hbm.at[sidx.at[pl.ds(i * CH, CH)]],
                              rows[b], gsem[b]).wait()

    def swait(b):
        pltpu.make_async_copy(rows[b], acc.at[didxs[b]], ssem[b]).wait()

    def scale_chunk(i, b):
        @plsc.parallel_loop(0, CH, unroll=16)
        def scale(k):
            w = plsc.load_gather(ewb, [jnp.full((L,), i * CH + k, jnp.int32)])
            for j in range(H // L):
                rows[b][k, pl.ds(j * L, L)] = rows[b][k, pl.ds(j * L, L)] * w

    for b in range(NBUF):
        gather(b, b)

    def body(g, _):
        for b in range(NBUF):
            i = g * NBUF + b
            gwait(i, b)
            scale_chunk(i, b)
            pltpu.async_copy(rows[b], acc.at[didxs[b]], ssem[b], add=True)

            @pl.when(i + NBUF < NCHUNK)
            def _():
                swait(b)
                gather(i + NBUF, b)
        return 0
    lax.fori_loop(0, NCHUNK // NBUF, body, 0)

    for b in range(NBUF):
        swait(b)

    plsc.subcore_barrier()
    pltpu.sync_copy(acc.at[pl.ds(s * STRIPE, STRIPE)],
                    out_hbm.at[c, pl.ds(s * STRIPE, STRIPE)])


_R = 1000


def _scale_body(d0_ref, d1_ref, emb_ref, xs_ref, dinv_ref):
    deg = d0_ref[...] + d1_ref[...] + 1.0
    dinv = lax.rsqrt(deg)
    dinv_ref[...] = dinv
    xs_ref[...] = emb_ref[...] * dinv


def _tc_scale(deg0, deg1, emb):
    return pl.pallas_call(
        _scale_body,
        grid=(N // _R,),
        in_specs=[
            pl.BlockSpec((_R, 1), lambda i: (i, 0)),
            pl.BlockSpec((_R, 1), lambda i: (i, 0)),
            pl.BlockSpec((_R, H), lambda i: (i, 0)),
        ],
        out_specs=[
            pl.BlockSpec((_R, H), lambda i: (i, 0)),
            pl.BlockSpec((_R, 1), lambda i: (i, 0)),
        ],
        out_shape=[
            jax.ShapeDtypeStruct((N, H), jnp.float32),
            jax.ShapeDtypeStruct((N, 1), jnp.float32),
        ],
    )(deg0, deg1, emb)


def _layer1_body(raw_ref, xs_ref, dinv_ref, w_ref, b_ref, out_ref):
    raw = raw_ref[...]
    agg = (raw[0] + raw[1] + xs_ref[...]) * dinv_ref[...]
    x1 = jnp.maximum(
        jnp.dot(agg, w_ref[...], preferred_element_type=jnp.float32)
        + b_ref[...], 0.0)
    out_ref[...] = x1 * dinv_ref[...]


def _tc_layer1(raw, xs0, dinv, W1, b1):
    return pl.pallas_call(
        _layer1_body,
        grid=(N // _R,),
        in_specs=[
            pl.BlockSpec((NC, _R, H), lambda i: (0, i, 0)),
            pl.BlockSpec((_R, H), lambda i: (i, 0)),
            pl.BlockSpec((_R, 1), lambda i: (i, 0)),
            pl.BlockSpec((H, H), lambda i: (0, 0)),
            pl.BlockSpec((1, H), lambda i: (0, 0)),
        ],
        out_specs=pl.BlockSpec((_R, H), lambda i: (i, 0)),
        out_shape=jax.ShapeDtypeStruct((N, H), jnp.float32),
    )(raw, xs0, dinv, W1, b1)


def _layer2_body(raw_ref, xs_ref, dinv_ref, wm_ref, bm_ref,
                 wl_ref, bl_ref, mu_ref, ls_ref):
    raw = raw_ref[...]
    agg = (raw[0] + raw[1] + xs_ref[...]) * dinv_ref[...]
    mu_ref[...] = jnp.dot(agg, wm_ref[...],
                          preferred_element_type=jnp.float32) + bm_ref[...]
    ls_ref[...] = jnp.dot(agg, wl_ref[...],
                          preferred_element_type=jnp.float32) + bl_ref[...]


def _tc_layer2(raw, xs1, dinv, Wmu, bmu, Wls, bls):
    return pl.pallas_call(
        _layer2_body,
        grid=(N // _R,),
        in_specs=[
            pl.BlockSpec((NC, _R, H), lambda i: (0, i, 0)),
            pl.BlockSpec((_R, H), lambda i: (i, 0)),
            pl.BlockSpec((_R, 1), lambda i: (i, 0)),
            pl.BlockSpec((H, O), lambda i: (0, 0)),
            pl.BlockSpec((1, O), lambda i: (0, 0)),
            pl.BlockSpec((H, O), lambda i: (0, 0)),
            pl.BlockSpec((1, O), lambda i: (0, 0)),
        ],
        out_specs=[
            pl.BlockSpec((_R, O), lambda i: (i, 0)),
            pl.BlockSpec((_R, O), lambda i: (i, 0)),
        ],
        out_shape=[
            jax.ShapeDtypeStruct((N, O), jnp.float32),
            jax.ShapeDtypeStruct((N, O), jnp.float32),
        ],
    )(raw, xs1, dinv, Wmu, bmu, Wls, bls)


def kernel(edge_index, edge_weight, emb, W1, b1, Wmu, bmu, Wls, bls):
    src = edge_index[0]
    dst = edge_index[1]

    degp = _deg_kernel(dst, edge_weight)
    deg0 = degp[0, :N].reshape(N, 1)
    deg1 = degp[1, :N].reshape(N, 1)

    xs0, dinv = _tc_scale(deg0, deg1, emb)

    raw0 = _spmm_kernel(src, dst, edge_weight, xs0)
    xs1 = _tc_layer1(raw0, xs0, dinv, W1, b1.reshape(1, H))

    raw1 = _spmm_kernel(src, dst, edge_weight, xs1)
    mu, logstd = _tc_layer2(raw1, xs1, dinv,
                            Wmu, bmu.reshape(1, O), Wls, bls.reshape(1, O))
    return (mu, logstd)


# speedup vs baseline: 1.3642x; 1.3642x over previous
import functools

import jax
import jax.numpy as jnp
from jax import lax
from jax.experimental import pallas as pl
from jax.experimental.pallas import tpu as pltpu
from jax.experimental.pallas import tpu_sc as plsc

N = 10000
H = 128
O = 64
E = 320000

NC = 2
NS = 16
L = 16

EPC = E // NC
EPT = EPC // NS
CH = 40
NCHUNK = EPT // CH
NBUF = 5

NPAD = 10240
STRIPE = NPAD // NS

_mesh = plsc.VectorSubcoreMesh(core_axis_name="c", subcore_axis_name="s",
                               num_cores=NC, num_subcores=NS)


@functools.partial(
    pl.kernel,
    out_type=jax.ShapeDtypeStruct((NC, NPAD), jnp.float32),
    mesh=_mesh,
    scratch_types=[
        pltpu.VMEM_SHARED((NPAD,), jnp.float32),
        pltpu.VMEM((EPT,), jnp.int32),
        pltpu.VMEM((EPT,), jnp.float32),
        pltpu.VMEM((STRIPE,), jnp.float32),
        pltpu.SemaphoreType.DMA,
    ],
)
def _deg_kernel(dst_hbm, ew_hbm, out_hbm, acc, idx_b, val_b, zbuf, sem):
    c = lax.axis_index("c")
    s = lax.axis_index("s")
    base = c * EPC + s * EPT

    pltpu.async_copy(dst_hbm.at[pl.ds(base, EPT)], idx_b, sem)
    pltpu.async_copy(ew_hbm.at[pl.ds(base, EPT)], val_b, sem)

    @plsc.parallel_loop(0, STRIPE // L, unroll=8)
    def zero_body(j):
        zbuf[pl.ds(j * L, L)] = jnp.zeros((L,), jnp.float32)
    pltpu.sync_copy(zbuf, acc.at[pl.ds(s * STRIPE, STRIPE)])

    pltpu.make_async_copy(dst_hbm.at[pl.ds(base, EPT)], idx_b, sem).wait()
    pltpu.make_async_copy(ew_hbm.at[pl.ds(base, EPT)], val_b, sem).wait()
    plsc.subcore_barrier()

    pltpu.sync_copy(val_b, acc.at[idx_b], add=True)

    plsc.subcore_barrier()
    pltpu.sync_copy(acc.at[pl.ds(s * STRIPE, STRIPE)],
                    out_hbm.at[c, pl.ds(s * STRIPE, STRIPE)])


@functools.partial(
    pl.kernel,
    out_type=jax.ShapeDtypeStruct((NC, NPAD, H), jnp.float32),
    mesh=_mesh,
    scratch_types=[
        pltpu.VMEM_SHARED((NPAD, H), jnp.float32),
        pltpu.VMEM((EPT,), jnp.int32),
        pltpu.VMEM((EPT,), jnp.float32),
        pltpu.VMEM((CH,), jnp.int32),
        pltpu.VMEM((CH,), jnp.int32),
        pltpu.VMEM((CH,), jnp.int32),
        pltpu.VMEM((CH,), jnp.int32),
        pltpu.VMEM((CH,), jnp.int32),
        pltpu.VMEM((CH, H), jnp.float32),
        pltpu.VMEM((CH, H), jnp.float32),
        pltpu.VMEM((CH, H), jnp.float32),
        pltpu.VMEM((CH, H), jnp.float32),
        pltpu.VMEM((CH, H), jnp.float32),
        pltpu.SemaphoreType.DMA,
        pltpu.SemaphoreType.DMA,
        pltpu.SemaphoreType.DMA,
        pltpu.SemaphoreType.DMA,
        pltpu.SemaphoreType.DMA,
        pltpu.SemaphoreType.DMA,
        pltpu.SemaphoreType.DMA,
        pltpu.SemaphoreType.DMA,
        pltpu.SemaphoreType.DMA,
        pltpu.SemaphoreType.DMA,
    ],
    compiler_params=pltpu.CompilerParams(needs_layout_passes=False),
)
def _spmm_kernel(src_hbm, dst_hbm, ew_hbm, xs_hbm, out_hbm,
                 acc, sidx, ewb, didx0, didx1, didx2, didx3, didx4,
                 rows0, rows1, rows2, rows3, rows4,
                 g0, g1, g2, g3, g4, s0, s1, s2, s3, s4):
    c = lax.axis_index("c")
    s = lax.axis_index("s")
    gsem = (g0, g1, g2, g3, g4)
    ssem = (s0, s1, s2, s3, s4)
    didxs = (didx0, didx1, didx2, didx3, didx4)
    rows = (rows0, rows1, rows2, rows3, rows4)

    base = c * EPC + s * EPT
    pltpu.async_copy(src_hbm.at[pl.ds(base, EPT)], sidx, g0)
    pltpu.async_copy(ew_hbm.at[pl.ds(base, EPT)], ewb, g1)

    @plsc.parallel_loop(0, CH * (H // L), unroll=8)
    def zrow(j):
        k = j // (H // L)
        col = (j % (H // L)) * L
        rows0[k, pl.ds(col, L)] = jnp.zeros((L,), jnp.float32)

    def zcopy(r, _):
        pltpu.async_copy(rows0.at[pl.ds(0, CH)],
                         acc.at[pl.ds(s * STRIPE + r * CH, CH)], s0)
        return 0
    lax.fori_loop(0, STRIPE // CH, zcopy, 0)

    def zdrain(r, _):
        pltpu.make_async_copy(rows0.at[pl.ds(0, CH)],
                              acc.at[pl.ds(s * STRIPE, CH)], s0).wait()
        return 0
    lax.fori_loop(0, STRIPE // CH, zdrain, 0)

    pltpu.make_async_copy(src_hbm.at[pl.ds(base, EPT)], sidx, g0).wait()
    pltpu.make_async_copy(ew_hbm.at[pl.ds(base, EPT)], ewb, g1).wait()
    plsc.subcore_barrier()

    def gather(i, b):
        pltpu.async_copy(dst_hbm.at[pl.ds(base + i * CH, CH)],
                         didxs[b], gsem[b])
        pltpu.async_copy(xs_hbm.at[sidx.at[pl.ds(i * CH, CH)]],
                         rows[b], gsem[b])

    def gwait(i, b):
        pltpu.make_async_copy(dst_hbm.at[pl.ds(base + i * CH, CH)],
                              didxs[b], gsem[b]).wait()
        pltpu.make_async_copy(xs_hbm.at[sidx.at[pl.ds(i * CH, CH)]],
                              rows[b], gsem[b]).wait()

    def swait(b):
        pltpu.make_async_copy(rows[b], acc.at[didxs[b]], ssem[b]).wait()

    def scale_chunk(i, b):
        @plsc.parallel_loop(0, CH, unroll=4)
        def scale(k):
            w = plsc.load_gather(ewb, [jnp.full((L,), i * CH + k, jnp.int32)])
            for j in range(H // L):
                rows[b][k, pl.ds(j * L, L)] = rows[b][k, pl.ds(j * L, L)] * w

    for b in range(NBUF):
        gather(b, b)

    def body(g, _):
        for b in range(NBUF):
            i = g * NBUF + b
            gwait(i, b)
            scale_chunk(i, b)
            pltpu.async_copy(rows[b], acc.at[didxs[b]], ssem[b], add=True)

            @pl.when(i + NBUF < NCHUNK)
            def _():
                swait(b)
                gather(i + NBUF, b)
        return 0
    lax.fori_loop(0, NCHUNK // NBUF, body, 0)

    for b in range(NBUF):
        swait(b)

    plsc.subcore_barrier()
    pltpu.sync_copy(acc.at[pl.ds(s * STRIPE, STRIPE)],
                    out_hbm.at[c, pl.ds(s * STRIPE, STRIPE)])


_R = 1000


def _scale_body(d0_ref, d1_ref, emb_ref, xs_ref, dinv_ref):
    deg = d0_ref[...] + d1_ref[...] + 1.0
    dinv = lax.rsqrt(deg)
    dinv_ref[...] = dinv
    xs_ref[...] = emb_ref[...] * dinv


def _tc_scale(deg0, deg1, emb):
    return pl.pallas_call(
        _scale_body,
        grid=(N // _R,),
        in_specs=[
            pl.BlockSpec((_R, 1), lambda i: (i, 0)),
            pl.BlockSpec((_R, 1), lambda i: (i, 0)),
            pl.BlockSpec((_R, H), lambda i: (i, 0)),
        ],
        out_specs=[
            pl.BlockSpec((_R, H), lambda i: (i, 0)),
            pl.BlockSpec((_R, 1), lambda i: (i, 0)),
        ],
        out_shape=[
            jax.ShapeDtypeStruct((N, H), jnp.float32),
            jax.ShapeDtypeStruct((N, 1), jnp.float32),
        ],
    )(deg0, deg1, emb)


def _layer1_body(raw_ref, xs_ref, dinv_ref, w_ref, b_ref, out_ref):
    raw = raw_ref[...]
    agg = (raw[0] + raw[1] + xs_ref[...]) * dinv_ref[...]
    x1 = jnp.maximum(
        jnp.dot(agg, w_ref[...], preferred_element_type=jnp.float32)
        + b_ref[...], 0.0)
    out_ref[...] = x1 * dinv_ref[...]


def _tc_layer1(raw, xs0, dinv, W1, b1):
    return pl.pallas_call(
        _layer1_body,
        grid=(N // _R,),
        in_specs=[
            pl.BlockSpec((NC, _R, H), lambda i: (0, i, 0)),
            pl.BlockSpec((_R, H), lambda i: (i, 0)),
            pl.BlockSpec((_R, 1), lambda i: (i, 0)),
            pl.BlockSpec((H, H), lambda i: (0, 0)),
            pl.BlockSpec((1, H), lambda i: (0, 0)),
        ],
        out_specs=pl.BlockSpec((_R, H), lambda i: (i, 0)),
        out_shape=jax.ShapeDtypeStruct((N, H), jnp.float32),
    )(raw, xs0, dinv, W1, b1)


def _layer2_body(raw_ref, xs_ref, dinv_ref, wm_ref, bm_ref,
                 wl_ref, bl_ref, mu_ref, ls_ref):
    raw = raw_ref[...]
    agg = (raw[0] + raw[1] + xs_ref[...]) * dinv_ref[...]
    mu_ref[...] = jnp.dot(agg, wm_ref[...],
                          preferred_element_type=jnp.float32) + bm_ref[...]
    ls_ref[...] = jnp.dot(agg, wl_ref[...],
                          preferred_element_type=jnp.float32) + bl_ref[...]


def _tc_layer2(raw, xs1, dinv, Wmu, bmu, Wls, bls):
    return pl.pallas_call(
        _layer2_body,
        grid=(N // _R,),
        in_specs=[
            pl.BlockSpec((NC, _R, H), lambda i: (0, i, 0)),
            pl.BlockSpec((_R, H), lambda i: (i, 0)),
            pl.BlockSpec((_R, 1), lambda i: (i, 0)),
            pl.BlockSpec((H, O), lambda i: (0, 0)),
            pl.BlockSpec((1, O), lambda i: (0, 0)),
            pl.BlockSpec((H, O), lambda i: (0, 0)),
            pl.BlockSpec((1, O), lambda i: (0, 0)),
        ],
        out_specs=[
            pl.BlockSpec((_R, O), lambda i: (i, 0)),
            pl.BlockSpec((_R, O), lambda i: (i, 0)),
        ],
        out_shape=[
            jax.ShapeDtypeStruct((N, O), jnp.float32),
            jax.ShapeDtypeStruct((N, O), jnp.float32),
        ],
    )(raw, xs1, dinv, Wmu, bmu, Wls, bls)


def kernel(edge_index, edge_weight, emb, W1, b1, Wmu, bmu, Wls, bls):
    src = edge_index[0]
    dst = edge_index[1]

    degp = _deg_kernel(dst, edge_weight)
    deg0 = degp[0, :N].reshape(N, 1)
    deg1 = degp[1, :N].reshape(N, 1)

    xs0, dinv = _tc_scale(deg0, deg1, emb)

    raw0 = _spmm_kernel(src, dst, edge_weight, xs0)
    xs1 = _tc_layer1(raw0, xs0, dinv, W1, b1.reshape(1, H))

    raw1 = _spmm_kernel(src, dst, edge_weight, xs1)
    mu, logstd = _tc_layer2(raw1, xs1, dinv,
                            Wmu, bmu.reshape(1, O), Wls, bls.reshape(1, O))
    return (mu, logstd)


# speedup vs baseline: 1.3717x; 1.0055x over previous
import functools

import jax
import jax.numpy as jnp
from jax import lax
from jax.experimental import pallas as pl
from jax.experimental.pallas import tpu as pltpu
from jax.experimental.pallas import tpu_sc as plsc

N = 10000
H = 128
O = 64
E = 320000

NC = 2
NS = 16
L = 16

EPC = E // NC
EPT = EPC // NS
CH = 40
NCHUNK = EPT // CH
NBUF = 5

NPAD = 10240
STRIPE = NPAD // NS

_mesh = plsc.VectorSubcoreMesh(core_axis_name="c", subcore_axis_name="s",
                               num_cores=NC, num_subcores=NS)


@functools.partial(
    pl.kernel,
    out_type=jax.ShapeDtypeStruct((NC, NPAD), jnp.float32),
    mesh=_mesh,
    scratch_types=[
        pltpu.VMEM_SHARED((NPAD,), jnp.float32),
        pltpu.VMEM((EPT,), jnp.int32),
        pltpu.VMEM((EPT,), jnp.float32),
        pltpu.VMEM((STRIPE,), jnp.float32),
        pltpu.SemaphoreType.DMA,
    ],
)
def _deg_kernel(dst_hbm, ew_hbm, out_hbm, acc, idx_b, val_b, zbuf, sem):
    c = lax.axis_index("c")
    s = lax.axis_index("s")
    base = c * EPC + s * EPT

    pltpu.async_copy(dst_hbm.at[pl.ds(base, EPT)], idx_b, sem)
    pltpu.async_copy(ew_hbm.at[pl.ds(base, EPT)], val_b, sem)

    @plsc.parallel_loop(0, STRIPE // L, unroll=8)
    def zero_body(j):
        zbuf[pl.ds(j * L, L)] = jnp.zeros((L,), jnp.float32)
    pltpu.sync_copy(zbuf, acc.at[pl.ds(s * STRIPE, STRIPE)])

    pltpu.make_async_copy(dst_hbm.at[pl.ds(base, EPT)], idx_b, sem).wait()
    pltpu.make_async_copy(ew_hbm.at[pl.ds(base, EPT)], val_b, sem).wait()
    plsc.subcore_barrier()

    pltpu.sync_copy(val_b, acc.at[idx_b], add=True)

    plsc.subcore_barrier()
    pltpu.sync_copy(acc.at[pl.ds(s * STRIPE, STRIPE)],
                    out_hbm.at[c, pl.ds(s * STRIPE, STRIPE)])


@functools.partial(
    pl.kernel,
    out_type=jax.ShapeDtypeStruct((NC, NPAD, H), jnp.float32),
    mesh=_mesh,
    scratch_types=[
        pltpu.VMEM_SHARED((NPAD, H), jnp.float32),
        pltpu.VMEM((EPT,), jnp.int32),
        pltpu.VMEM((EPT,), jnp.float32),
        pltpu.VMEM((CH,), jnp.int32),
        pltpu.VMEM((CH,), jnp.int32),
        pltpu.VMEM((CH,), jnp.int32),
        pltpu.VMEM((CH,), jnp.int32),
        pltpu.VMEM((CH,), jnp.int32),
        pltpu.VMEM((CH, H), jnp.float32),
        pltpu.VMEM((CH, H), jnp.float32),
        pltpu.VMEM((CH, H), jnp.float32),
        pltpu.VMEM((CH, H), jnp.float32),
        pltpu.VMEM((CH, H), jnp.float32),
        pltpu.SemaphoreType.DMA,
        pltpu.SemaphoreType.DMA,
        pltpu.SemaphoreType.DMA,
        pltpu.SemaphoreType.DMA,
        pltpu.SemaphoreType.DMA,
        pltpu.SemaphoreType.DMA,
        pltpu.SemaphoreType.DMA,
        pltpu.SemaphoreType.DMA,
        pltpu.SemaphoreType.DMA,
        pltpu.SemaphoreType.DMA,
    ],
    compiler_params=pltpu.CompilerParams(needs_layout_passes=False),
)
def _spmm_kernel(src_hbm, dst_hbm, ew_hbm, xs_hbm, out_hbm,
                 acc, sidx, ewb, didx0, didx1, didx2, didx3, didx4,
                 rows0, rows1, rows2, rows3, rows4,
                 g0, g1, g2, g3, g4, s0, s1, s2, s3, s4):
    c = lax.axis_index("c")
    s = lax.axis_index("s")
    gsem = (g0, g1, g2, g3, g4)
    ssem = (s0, s1, s2, s3, s4)
    didxs = (didx0, didx1, didx2, didx3, didx4)
    rows = (rows0, rows1, rows2, rows3, rows4)

    base = c * EPC + s * EPT
    pltpu.async_copy(src_hbm.at[pl.ds(base, EPT)], sidx, g0)
    pltpu.async_copy(ew_hbm.at[pl.ds(base, EPT)], ewb, g1)

    @plsc.parallel_loop(0, CH * (H // L), unroll=8)
    def zrow(j):
        k = j // (H // L)
        col = (j % (H // L)) * L
        rows0[k, pl.ds(col, L)] = jnp.zeros((L,), jnp.float32)

    def zcopy(r, _):
        pltpu.async_copy(rows0.at[pl.ds(0, CH)],
                         acc.at[pl.ds(s * STRIPE + r * CH, CH)], s0)
        return 0
    lax.fori_loop(0, STRIPE // CH, zcopy, 0)

    pltpu.make_async_copy(src_hbm.at[pl.ds(base, EPT)], sidx, g0).wait()
    pltpu.make_async_copy(ew_hbm.at[pl.ds(base, EPT)], ewb, g1).wait()

    def gather(i, b):
        pltpu.async_copy(dst_hbm.at[pl.ds(base + i * CH, CH)],
                         didxs[b], gsem[b])
        pltpu.async_copy(xs_hbm.at[sidx.at[pl.ds(i * CH, CH)]],
                         rows[b], gsem[b])

    def gwait(i, b):
        pltpu.make_async_copy(dst_hbm.at[pl.ds(base + i * CH, CH)],
                              didxs[b], gsem[b]).wait()
        pltpu.make_async_copy(xs_hbm.at[sidx.at[pl.ds(i * CH, CH)]],
                              rows[b], gsem[b]).wait()

    def swait(b):
        pltpu.make_async_copy(rows[b], acc.at[didxs[b]], ssem[b]).wait()

    def scale_chunk(i, b):
        @plsc.parallel_loop(0, CH, unroll=4)
        def scale(k):
            w = plsc.load_gather(ewb, [jnp.full((L,), i * CH + k, jnp.int32)])
            for j in range(H // L):
                rows[b][k, pl.ds(j * L, L)] = rows[b][k, pl.ds(j * L, L)] * w

    for b in range(1, NBUF):
        gather(b, b)

    def zdrain(r, _):
        pltpu.make_async_copy(rows0.at[pl.ds(0, CH)],
                              acc.at[pl.ds(s * STRIPE, CH)], s0).wait()
        return 0
    lax.fori_loop(0, STRIPE // CH, zdrain, 0)
    gather(0, 0)
    plsc.subcore_barrier()

    def body(g, _):
        for b in range(NBUF):
            i = g * NBUF + b
            gwait(i, b)
            scale_chunk(i, b)
            pltpu.async_copy(rows[b], acc.at[didxs[b]], ssem[b], add=True)

            @pl.when(i + NBUF < NCHUNK)
            def _():
                swait(b)
                gather(i + NBUF, b)
        return 0
    lax.fori_loop(0, NCHUNK // NBUF, body, 0)

    for b in range(NBUF):
        swait(b)

    plsc.subcore_barrier()
    pltpu.sync_copy(acc.at[pl.ds(s * STRIPE, STRIPE)],
                    out_hbm.at[c, pl.ds(s * STRIPE, STRIPE)])


_R = 1000


def _scale_body(d0_ref, d1_ref, emb_ref, xs_ref, dinv_ref):
    deg = d0_ref[...] + d1_ref[...] + 1.0
    dinv = lax.rsqrt(deg)
    dinv_ref[...] = dinv
    xs_ref[...] = emb_ref[...] * dinv


def _tc_scale(deg0, deg1, emb):
    return pl.pallas_call(
        _scale_body,
        grid=(N // _R,),
        in_specs=[
            pl.BlockSpec((_R, 1), lambda i: (i, 0)),
            pl.BlockSpec((_R, 1), lambda i: (i, 0)),
            pl.BlockSpec((_R, H), lambda i: (i, 0)),
        ],
        out_specs=[
            pl.BlockSpec((_R, H), lambda i: (i, 0)),
            pl.BlockSpec((_R, 1), lambda i: (i, 0)),
        ],
        out_shape=[
            jax.ShapeDtypeStruct((N, H), jnp.float32),
            jax.ShapeDtypeStruct((N, 1), jnp.float32),
        ],
    )(deg0, deg1, emb)


def _layer1_body(raw_ref, xs_ref, dinv_ref, w_ref, b_ref, out_ref):
    raw = raw_ref[...]
    agg = (raw[0] + raw[1] + xs_ref[...]) * dinv_ref[...]
    x1 = jnp.maximum(
        jnp.dot(agg, w_ref[...], preferred_element_type=jnp.float32)
        + b_ref[...], 0.0)
    out_ref[...] = x1 * dinv_ref[...]


def _tc_layer1(raw, xs0, dinv, W1, b1):
    return pl.pallas_call(
        _layer1_body,
        grid=(N // _R,),
        in_specs=[
            pl.BlockSpec((NC, _R, H), lambda i: (0, i, 0)),
            pl.BlockSpec((_R, H), lambda i: (i, 0)),
            pl.BlockSpec((_R, 1), lambda i: (i, 0)),
            pl.BlockSpec((H, H), lambda i: (0, 0)),
            pl.BlockSpec((1, H), lambda i: (0, 0)),
        ],
        out_specs=pl.BlockSpec((_R, H), lambda i: (i, 0)),
        out_shape=jax.ShapeDtypeStruct((N, H), jnp.float32),
    )(raw, xs0, dinv, W1, b1)


def _layer2_body(raw_ref, xs_ref, dinv_ref, wm_ref, bm_ref,
                 wl_ref, bl_ref, mu_ref, ls_ref):
    raw = raw_ref[...]
    agg = (raw[0] + raw[1] + xs_ref[...]) * dinv_ref[...]
    mu_ref[...] = jnp.dot(agg, wm_ref[...],
                          preferred_element_type=jnp.float32) + bm_ref[...]
    ls_ref[...] = jnp.dot(agg, wl_ref[...],
                          preferred_element_type=jnp.float32) + bl_ref[...]


def _tc_layer2(raw, xs1, dinv, Wmu, bmu, Wls, bls):
    return pl.pallas_call(
        _layer2_body,
        grid=(N // _R,),
        in_specs=[
            pl.BlockSpec((NC, _R, H), lambda i: (0, i, 0)),
            pl.BlockSpec((_R, H), lambda i: (i, 0)),
            pl.BlockSpec((_R, 1), lambda i: (i, 0)),
            pl.BlockSpec((H, O), lambda i: (0, 0)),
            pl.BlockSpec((1, O), lambda i: (0, 0)),
            pl.BlockSpec((H, O), lambda i: (0, 0)),
            pl.BlockSpec((1, O), lambda i: (0, 0)),
        ],
        out_specs=[
            pl.BlockSpec((_R, O), lambda i: (i, 0)),
            pl.BlockSpec((_R, O), lambda i: (i, 0)),
        ],
        out_shape=[
            jax.ShapeDtypeStruct((N, O), jnp.float32),
            jax.ShapeDtypeStruct((N, O), jnp.float32),
        ],
    )(raw, xs1, dinv, Wmu, bmu, Wls, bls)


def kernel(edge_index, edge_weight, emb, W1, b1, Wmu, bmu, Wls, bls):
    src = edge_index[0]
    dst = edge_index[1]

    degp = _deg_kernel(dst, edge_weight)
    deg0 = degp[0, :N].reshape(N, 1)
    deg1 = degp[1, :N].reshape(N, 1)

    xs0, dinv = _tc_scale(deg0, deg1, emb)

    raw0 = _spmm_kernel(src, dst, edge_weight, xs0)
    xs1 = _tc_layer1(raw0, xs0, dinv, W1, b1.reshape(1, H))

    raw1 = _spmm_kernel(src, dst, edge_weight, xs1)
    mu, logstd = _tc_layer2(raw1, xs1, dinv,
                            Wmu, bmu.reshape(1, O), Wls, bls.reshape(1, O))
    return (mu, logstd)
